# baseline (speedup 1.0000x reference)
import jax
import jax.numpy as jnp
from jax.experimental import pallas as pl
from jax.experimental.pallas import tpu as pltpu

N = 2073600
FEAT_DIM = 12
K = 32
G = 8
BLK = 25600
BQ = BLK // G


def _fused_body(fp_ref, rp_ref, cb_ref, cbw1t_ref, w1rt_ref, b1_ref,
                w2t_ref, b2_ref, w3t_ref, b3_ref, out_ref):
    cb = cb_ref[...]
    cb_half_sq = 0.5 * jnp.sum(cb * cb, axis=1)[:, None]
    ft = fp_ref[...].T
    rt = rp_ref[...].T

    pieces = []
    for a in range(G):
        fa = ft[FEAT_DIM * a:FEAT_DIM * (a + 1), :]
        ra = rt[3 * a:3 * a + 3, :]

        scores = jnp.dot(cb, fa, preferred_element_type=jnp.float32) - cb_half_sq
        m = jnp.max(scores, axis=0, keepdims=True)
        ii = jax.lax.broadcasted_iota(jnp.int32, scores.shape, 0)
        masked_ii = jnp.where(scores >= m, ii, K)
        amin = jnp.min(masked_ii, axis=0, keepdims=True)
        one_hot = (ii == amin).astype(jnp.float32)

        h = (jnp.dot(cbw1t_ref[...], one_hot, preferred_element_type=jnp.float32)
             + jnp.dot(w1rt_ref[...], ra, preferred_element_type=jnp.float32)
             + b1_ref[...])
        h = jnp.maximum(h, 0.0)
        h = jnp.dot(w2t_ref[...], h, preferred_element_type=jnp.float32) + b2_ref[...]
        h = jnp.maximum(h, 0.0)
        o = jnp.dot(w3t_ref[...], h, preferred_element_type=jnp.float32) + b3_ref[...]
        o = jnp.clip(jax.nn.sigmoid(o), 0.0, 1.0)
        pieces.append(o.T[:, None, :])

    out_ref[...] = jnp.concatenate(pieces, axis=1).reshape(BLK, 3)


@jax.jit
def _run(fp, rp, codebook, cbw1t, w1rt, b1, w2t, b2, w3t, b3):
    rep = lambda shape: pl.BlockSpec(shape, lambda i: tuple(0 for _ in shape))
    return pl.pallas_call(
        _fused_body,
        grid=(N // BLK,),
        in_specs=[
            pl.BlockSpec((BQ, G * FEAT_DIM), lambda i: (i, 0)),
            pl.BlockSpec((BQ, G * 3), lambda i: (i, 0)),
            rep((K, FEAT_DIM)),
            rep((K, K)),
            rep((K, 3)),
            rep((K, 1)),
            rep((K, K)),
            rep((K, 1)),
            rep((3, K)),
            rep((3, 1)),
        ],
        out_specs=pl.BlockSpec((BLK, 3), lambda i: (i, 0)),
        out_shape=jax.ShapeDtypeStruct((N, 3), jnp.float32),
        compiler_params=pltpu.CompilerParams(
            dimension_semantics=("arbitrary",),
        ),
    )(fp, rp, codebook, cbw1t, w1rt, b1, w2t, b2, w3t, b3)


def kernel(feat_enc, rays_d, codebook, W1, b1, W2, b2, W3, b3):
    fp = feat_enc.reshape(N // G, G * FEAT_DIM)
    rp = rays_d.reshape(N // G, G * 3)
    cbw1t = (codebook @ W1[:FEAT_DIM]).T
    return _run(fp, rp, codebook, cbw1t, W1[FEAT_DIM:].T,
                b1.reshape(K, 1), W2.T, b2.reshape(K, 1),
                W3.T, b3.reshape(3, 1))


# speedup vs baseline: 3.2425x; 3.2425x over previous
import functools

import jax
import jax.numpy as jnp
from jax.experimental import pallas as pl
from jax.experimental.pallas import tpu as pltpu

N = 2073600
FEAT_DIM = 12
K = 32
BLK = 6400


def _fused_body(feat_ref, rays_ref, cb_ref, w1f_ref, w1r_ref, b1_ref,
                w2_ref, b2_ref, w3_ref, b3_ref, out_ref):
    f = feat_ref[...]
    r = rays_ref[...]
    cb = cb_ref[...]

    cb_half_sq = 0.5 * jnp.sum(cb * cb, axis=1)[None, :]
    scores = jax.lax.dot_general(
        f, cb, (((1,), (1,)), ((), ())),
        preferred_element_type=jnp.float32) - cb_half_sq

    m = jnp.max(scores, axis=1, keepdims=True)
    ii = jax.lax.broadcasted_iota(jnp.int32, scores.shape, 1)
    masked_ii = jnp.where(scores >= m, ii, K)
    amin = jnp.min(masked_ii, axis=1, keepdims=True)
    one_hot = (ii == amin).astype(jnp.float32)

    cb_w1 = jnp.dot(cb, w1f_ref[...], preferred_element_type=jnp.float32)
    h = (jnp.dot(one_hot, cb_w1, preferred_element_type=jnp.float32)
         + jnp.dot(r, w1r_ref[...], preferred_element_type=jnp.float32)
         + b1_ref[...])
    h = jnp.maximum(h, 0.0)

    h = jnp.dot(h, w2_ref[...], preferred_element_type=jnp.float32) + b2_ref[...]
    h = jnp.maximum(h, 0.0)

    o = jnp.dot(h, w3_ref[...], preferred_element_type=jnp.float32) + b3_ref[...]
    o = jax.nn.sigmoid(o)
    out_ref[...] = jnp.clip(o, 0.0, 1.0)


@jax.jit
def _run(feat_enc, rays_d, codebook, W1f, W1r, b1, W2, b2, W3, b3):
    grid = (N // BLK,)
    blk = lambda shape: pl.BlockSpec((BLK,) + shape, lambda i: (i, 0))
    rep = lambda shape: pl.BlockSpec(shape, lambda i: (0, 0))
    return pl.pallas_call(
        _fused_body,
        grid=grid,
        in_specs=[
            blk((FEAT_DIM,)),
            blk((3,)),
            rep((K, FEAT_DIM)),
            rep((FEAT_DIM, 32)),
            rep((3, 32)),
            rep((1, 32)),
            rep((32, 32)),
            rep((1, 32)),
            rep((32, 3)),
            rep((1, 3)),
        ],
        out_specs=blk((3,)),
        out_shape=jax.ShapeDtypeStruct((N, 3), jnp.float32),
        compiler_params=pltpu.CompilerParams(
            dimension_semantics=("arbitrary",),
        ),
    )(feat_enc, rays_d, codebook, W1f, W1r, b1, W2, b2, W3, b3)


def kernel(feat_enc, rays_d, codebook, W1, b1, W2, b2, W3, b3):
    W1f = W1[:FEAT_DIM]
    W1r = W1[FEAT_DIM:]
    return _run(feat_enc, rays_d, codebook, W1f, W1r,
                b1.reshape(1, 32), W2, b2.reshape(1, 32), W3, b3.reshape(1, 3))


# speedup vs baseline: 3.3609x; 1.0365x over previous
import functools

import jax
import jax.numpy as jnp
from jax.experimental import pallas as pl
from jax.experimental.pallas import tpu as pltpu

N = 2073600
FEAT_DIM = 12
K = 32
BLK = 12800


def _fused_body(feat_ref, rays_ref, cb_ref, w1f_ref, w1r_ref, b1_ref,
                w2_ref, b2_ref, w3_ref, b3_ref, out_ref):
    f = feat_ref[...]
    r = rays_ref[...]
    cb = cb_ref[...]

    cb_half_sq = 0.5 * jnp.sum(cb * cb, axis=1)[None, :]
    scores = jax.lax.dot_general(
        f, cb, (((1,), (1,)), ((), ())),
        preferred_element_type=jnp.float32) - cb_half_sq

    m = jnp.max(scores, axis=1, keepdims=True)
    ii = jax.lax.broadcasted_iota(jnp.int32, scores.shape, 1)
    masked_ii = jnp.where(scores >= m, ii, K)
    amin = jnp.min(masked_ii, axis=1, keepdims=True)
    one_hot = (ii == amin).astype(jnp.float32)

    cb_w1 = jnp.dot(cb, w1f_ref[...], preferred_element_type=jnp.float32)
    h = (jnp.dot(one_hot, cb_w1, preferred_element_type=jnp.float32)
         + jnp.dot(r, w1r_ref[...], preferred_element_type=jnp.float32)
         + b1_ref[...])
    h = jnp.maximum(h, 0.0)

    h = jnp.dot(h, w2_ref[...], preferred_element_type=jnp.float32) + b2_ref[...]
    h = jnp.maximum(h, 0.0)

    o = jnp.dot(h, w3_ref[...], preferred_element_type=jnp.float32) + b3_ref[...]
    o = jax.nn.sigmoid(o)
    out_ref[...] = jnp.clip(o, 0.0, 1.0)


@jax.jit
def _run(feat_enc, rays_d, codebook, W1f, W1r, b1, W2, b2, W3, b3):
    grid = (N // BLK,)
    blk = lambda shape: pl.BlockSpec((BLK,) + shape, lambda i: (i, 0))
    rep = lambda shape: pl.BlockSpec(shape, lambda i: (0, 0))
    return pl.pallas_call(
        _fused_body,
        grid=grid,
        in_specs=[
            blk((FEAT_DIM,)),
            blk((3,)),
            rep((K, FEAT_DIM)),
            rep((FEAT_DIM, 32)),
            rep((3, 32)),
            rep((1, 32)),
            rep((32, 32)),
            rep((1, 32)),
            rep((32, 3)),
            rep((1, 3)),
        ],
        out_specs=blk((3,)),
        out_shape=jax.ShapeDtypeStruct((N, 3), jnp.float32),
        compiler_params=pltpu.CompilerParams(
            dimension_semantics=("arbitrary",),
        ),
    )(feat_enc, rays_d, codebook, W1f, W1r, b1, W2, b2, W3, b3)


def kernel(feat_enc, rays_d, codebook, W1, b1, W2, b2, W3, b3):
    W1f = W1[:FEAT_DIM]
    W1r = W1[FEAT_DIM:]
    return _run(feat_enc, rays_d, codebook, W1f, W1r,
                b1.reshape(1, 32), W2, b2.reshape(1, 32), W3, b3.reshape(1, 3))
